# Initial kernel scaffold; baseline (speedup 1.0000x reference)
import jax
import jax.numpy as jnp
from jax.experimental import pallas as pl


def kernel(vel, pos0, pos1, W, b):
    raise NotImplementedError("write your pallas kernel here")


# speedup vs baseline: 70.8583x; 70.8583x over previous
import numpy as np

import jax
import jax.numpy as jnp
from jax import lax
from jax.experimental import pallas as pl
from jax.experimental.pallas import tpu as pltpu
from jax.experimental.pallas import tpu_sc as plsc

N = 20000
NQ = 8000
EXT = float(1.0 * 6 * 0.025)
RAD = EXT / 2.0
R2 = float(np.float32(RAD * RAD))
INVR = float(np.float32(2.0 / EXT))

NCELL = 13
OWN = 256
NCPAD = 16 * OWN
CHK = 1280
NPAD = 16 * CHK
NCAP = 4096
NSORT = N + 160
RODMAX = 128
RODCP = 136
K = 128
NQPAD = 8192
QPT = NQPAD // 32
QB = 400

_i32 = jnp.int32
_f32 = jnp.float32


def _sc_body(pdata, qpos, nbr_out, cnt_out,
             sp_cid, sp_cnt, sp_sorted,
             posb, cidall, hitcid, hitpid, perm2d, srows, wo, hist, histv,
             cntf, starts, qposb, qcell, rodbuf, qbuf, cntq, sem):
    s = lax.axis_index("s")
    c = lax.axis_index("c")
    wid = c * 16 + s
    iota = lax.iota(_i32, 16)
    zcol = jnp.zeros((16,), _i32)
    lane0 = iota == 0

    pltpu.sync_copy(pdata.at[pl.ds(s * CHK, CHK)], posb)

    def ph0(j, _):
        rowv = j * 16 + iota
        mrow = (s * CHK + rowv) < N
        rs = jnp.where(mrow, rowv, 0)
        px = plsc.load_gather(posb, [rs, zcol])
        py = plsc.load_gather(posb, [rs, zcol + 1])
        pz = plsc.load_gather(posb, [rs, zcol + 2])
        cx = jnp.clip((px * _f32(NCELL)).astype(_i32), 0, NCELL - 1)
        cy = jnp.clip((py * _f32(NCELL)).astype(_i32), 0, NCELL - 1)
        cz = jnp.clip((pz * _f32(NCELL)).astype(_i32), 0, NCELL - 1)
        cid = (cz * NCELL + cy) * NCELL + cx
        cid = jnp.where(mrow, cid, NCPAD + 99)
        cidall[pl.ds(j * 16, 16)] = cid
        return 0

    lax.fori_loop(0, CHK // 16, ph0, 0)
    pltpu.sync_copy(cidall.at[pl.ds(0, CHK)], sp_cid.at[pl.ds(s * CHK, CHK)])
    plsc.subcore_barrier()

    pltpu.sync_copy(sp_cid, cidall)
    lo = s * OWN

    def ph1(t, off):
        cv = cidall[pl.ds(t * 16, 16)]
        hit = (cv >= lo) & (cv < lo + OWN)
        hi32 = hit.astype(_i32)
        pos = off + plsc.cumsum(hi32) - hi32
        poss = jnp.minimum(pos, NCAP - 1)
        pidx = t * 16 + iota
        plsc.store_scatter(hitcid, [poss], cv, mask=hit)
        plsc.store_scatter(hitpid, [poss], pidx, mask=hit)
        return jnp.minimum(off + jnp.sum(hi32), NCAP)

    nhits = lax.fori_loop(0, NPAD // 16, ph1, 0)

    def zh(j, _):
        hist[j] = 0
        return 0

    lax.fori_loop(0, OWN, zh, 0)

    def ph1b(h, _):
        cl = hitcid[pl.ds(h, 16)][0] - lo
        n0 = hist[cl]
        hist[cl] = n0 + 1
        plsc.store_scatter(wo, [h + iota], jnp.broadcast_to(n0, (16,)),
                           mask=lane0)
        return 0

    lax.fori_loop(0, nhits, ph1b, 0)

    def hcp(j, _):
        hv = jnp.zeros((16,), _i32)
        for u in range(16):
            hv = jnp.where(iota == u, hist[j * 16 + u], hv)
        histv[pl.ds(j * 16, 16)] = hv
        return 0

    lax.fori_loop(0, OWN // 16, hcp, 0)
    pltpu.sync_copy(histv, sp_cnt.at[pl.ds(s * OWN, OWN)])
    plsc.subcore_barrier()

    pltpu.sync_copy(sp_cnt, cntf)

    def ph2(k, carry):
        v = cntf[pl.ds(k * 16, 16)]
        inc = plsc.cumsum(v)
        starts[pl.ds(k * 16, 16)] = carry + inc - v
        return carry + jnp.sum(v)

    total = lax.fori_loop(0, NCPAD // 16, ph2, 0)
    starts[pl.ds(NCPAD, 16)] = jnp.broadcast_to(total, (16,))
    start_lo = starts[pl.ds(lo, 16)][0]

    def pfz(j, _):
        hv = j * 16 + iota
        plsc.store_scatter(perm2d, [hv // 128, hv % 128], zcol)
        return 0

    lax.fori_loop(0, NCAP // 16, pfz, 0)

    def ph2b(j, _):
        hv = j * 16 + iota
        mh = hv < nhits
        hs = jnp.where(mh, hv, 0)
        cidv = plsc.load_gather(hitcid, [hs])
        wov = plsc.load_gather(wo, [hs])
        pidv = plsc.load_gather(hitpid, [hs])
        stv = plsc.load_gather(starts, [jnp.clip(cidv, 0, NCPAD - 1)])
        slot = stv + wov - start_lo
        oks = mh & (slot >= 0) & (slot < NCAP)
        slot = jnp.clip(slot, 0, NCAP - 1)
        plsc.store_scatter(perm2d, [slot // 128, slot % 128], pidv, mask=oks)
        return 0

    lax.fori_loop(0, NCAP // 16, ph2b, 0)

    for chunk in range(NCAP // 128):
        pltpu.async_copy(pdata.at[perm2d.at[chunk]],
                         srows.at[pl.ds(chunk * 128, 128)], sem).wait()

    nch = (nhits + 127) // 128
    shmax = jnp.maximum(nhits - 128, 0)

    def ph2c(j, _):
        offj = jnp.minimum(j * 128, shmax)
        pltpu.sync_copy(srows.at[pl.ds(offj, 128)],
                        sp_sorted.at[pl.ds(start_lo + offj, 128)])
        return 0

    lax.fori_loop(0, nch, ph2c, 0)
    plsc.subcore_barrier()

    qbase = wid * QPT
    pltpu.sync_copy(qpos.at[pl.ds(qbase, QPT)], qposb)

    def ph3cell(j, _):
        rowv = j * 16 + iota
        qx = plsc.load_gather(qposb, [rowv, zcol])
        qy = plsc.load_gather(qposb, [rowv, zcol + 1])
        qz = plsc.load_gather(qposb, [rowv, zcol + 2])
        ccx = jnp.clip((qx * _f32(NCELL)).astype(_i32), 0, NCELL - 1)
        ccy = jnp.clip((qy * _f32(NCELL)).astype(_i32), 0, NCELL - 1)
        ccz = jnp.clip((qz * _f32(NCELL)).astype(_i32), 0, NCELL - 1)
        plsc.store_scatter(qcell, [rowv * 8 + 0], ccx)
        plsc.store_scatter(qcell, [rowv * 8 + 1], ccy)
        plsc.store_scatter(qcell, [rowv * 8 + 2], ccz)
        plsc.store_scatter(qcell, [rowv * 8 + 4], plsc.bitcast(qx, _i32))
        plsc.store_scatter(qcell, [rowv * 8 + 5], plsc.bitcast(qy, _i32))
        plsc.store_scatter(qcell, [rowv * 8 + 6], plsc.bitcast(qz, _i32))
        return 0

    lax.fori_loop(0, QPT // 16, ph3cell, 0)

    def _bf16(x):
        u = plsc.bitcast(x, _i32)
        u2 = (u + 32767 + ((u >> 16) & 1)) & (-65536)
        return plsc.bitcast(u2, _f32)

    def ph3(qi, _):
        crow = qcell[pl.ds(qi * 8, 16)]
        frow = plsc.bitcast(crow, _f32)
        cx = crow[0]
        cy = crow[1]
        cz = crow[2]
        qx = frow[4]
        qy = frow[5]
        qz = frow[6]
        qq = (qx * qx + qy * qy) + qz * qz
        qxb = _bf16(jnp.broadcast_to(qx, (16,)))
        qyb = _bf16(jnp.broadcast_to(qy, (16,)))
        qzb = _bf16(jnp.broadcast_to(qz, (16,)))
        xlo = jnp.maximum(cx - 1, 0)
        xhi = jnp.minimum(cx + 1, NCELL - 1)
        off0 = 0

        def rod(off, dy, dz):
            cy2 = cy + dy
            cz2 = cz + dz
            okrod = (cy2 >= 0) & (cy2 < NCELL) & (cz2 >= 0) & (cz2 < NCELL)
            base = (cz2 * NCELL + cy2) * NCELL
            clo = jnp.clip(base + xlo, 0, NCPAD - 1)
            chi = jnp.clip(base + xhi, 0, NCPAD - 1)
            stt = jnp.clip(starts[pl.ds(clo, 16)][0], 0, N)
            end = starts[pl.ds(chi + 1, 16)][0]
            stt_al = stt & ~7
            skip = stt - stt_al
            n = jnp.where(okrod, jnp.minimum(end - stt, RODMAX), 0)

            @pl.when(n > 0)
            def _():
                pltpu.sync_copy(sp_sorted.at[pl.ds(stt_al, RODCP)], rodbuf)

            def chunkbody(j, off2):
                lv = j * 16 + iota
                mn = (lv >= skip) & (lv < skip + n)
                ls = jnp.where(mn, lv, 0)
                px = plsc.load_gather(rodbuf, [ls, zcol])
                py = plsc.load_gather(rodbuf, [ls, zcol + 1])
                pz = plsc.load_gather(rodbuf, [ls, zcol + 2])
                vx = plsc.load_gather(rodbuf, [ls, zcol + 3])
                vy = plsc.load_gather(rodbuf, [ls, zcol + 4])
                vz = plsc.load_gather(rodbuf, [ls, zcol + 5])
                ppv = plsc.load_gather(rodbuf, [ls, zcol + 6])
                ddx = px - qx
                ddy = py - qy
                ddz = pz - qz
                mqp = (qxb * _bf16(px) + qyb * _bf16(py)) + qzb * _bf16(pz)
                d2m = jnp.maximum((qq + ppv) - 2.0 * mqp, 0.0)
                ok = mn & (d2m <= R2) & (d2m > 0.0)
                oki = ok.astype(_i32)
                pos = jnp.minimum(off2 + plsc.cumsum(oki) - oki, K - 1)
                plsc.store_scatter(qbuf, [zcol, pos], ddx, mask=ok)
                plsc.store_scatter(qbuf, [zcol + 1, pos], ddy, mask=ok)
                plsc.store_scatter(qbuf, [zcol + 2, pos], ddz, mask=ok)
                plsc.store_scatter(qbuf, [zcol + 3, pos], vx, mask=ok)
                plsc.store_scatter(qbuf, [zcol + 4, pos], vy, mask=ok)
                plsc.store_scatter(qbuf, [zcol + 5, pos], vz, mask=ok)
                return jnp.minimum(off2 + jnp.sum(oki), K)

            nch = jnp.where(n > 0, (skip + n + 15) // 16, 0)
            return lax.fori_loop(0, nch, chunkbody, off)

        for dy in (-1, 0, 1):
            for dz in (-1, 0, 1):
                off0 = rod(off0, dy, dz)

        plsc.store_scatter(cntq, [qi + iota],
                           jnp.broadcast_to(off0, (16,)), mask=lane0)
        pltpu.sync_copy(qbuf, nbr_out.at[qbase + qi])
        return 0

    lax.fori_loop(0, QPT, ph3, 0)
    pltpu.sync_copy(cntq.at[pl.ds(0, QPT)], cnt_out.at[pl.ds(qbase, QPT)])


def _sc_stage(pdata, qpos):
    mesh = plsc.VectorSubcoreMesh(core_axis_name="c", subcore_axis_name="s",
                                  num_cores=2, num_subcores=16)
    return pl.kernel(
        _sc_body,
        out_type=[
            jax.ShapeDtypeStruct((NQPAD, 8, K), _f32),
            jax.ShapeDtypeStruct((NQPAD,), _i32),
        ],
        mesh=mesh,
        compiler_params=pltpu.CompilerParams(use_tc_tiling_on_sc=False,
                                             needs_layout_passes=False),
        scratch_types=[
            pltpu.VMEM_SHARED((NPAD,), _i32),
            pltpu.VMEM_SHARED((NCPAD,), _i32),
            pltpu.VMEM_SHARED((NSORT, 8), _f32),
            pltpu.VMEM((CHK, 8), _f32),
            pltpu.VMEM((NPAD,), _i32),
            pltpu.VMEM((NCAP + 16,), _i32),
            pltpu.VMEM((NCAP + 16,), _i32),
            pltpu.VMEM((NCAP // 128, 128), _i32),
            pltpu.VMEM((NCAP, 8), _f32),
            pltpu.VMEM((NCAP + 16,), _i32),
            pltpu.SMEM((OWN,), _i32),
            pltpu.VMEM((OWN,), _i32),
            pltpu.VMEM((NCPAD,), _i32),
            pltpu.VMEM((NCPAD + 32,), _i32),
            pltpu.VMEM((QPT, 4), _f32),
            pltpu.VMEM((QPT * 8 + 16,), _i32),
            pltpu.VMEM((RODCP, 8), _f32),
            pltpu.VMEM((8, K), _f32),
            pltpu.VMEM((QPT + 16,), _i32),
            pltpu.SemaphoreType.DMA,
        ],
    )(pdata, qpos)


def _tc_body(nbr_ref, cnt_ref, wf_ref, out_ref, feat_ref):
    nb = nbr_ref[...]
    cnt = cnt_ref[...]
    lane = lax.broadcasted_iota(_i32, (QB, K), 1)
    m = lane < cnt
    dx = jnp.where(m, nb[:, 0, :] * INVR, 2.0)
    dy = jnp.where(m, nb[:, 1, :] * INVR, 0.0)
    dz = jnp.where(m, nb[:, 2, :] * INVR, 0.0)
    rsq = dx * dx + dy * dy + dz * dz
    t = 1.0 - rsq
    win = jnp.clip(t * t * t, 0.0, 1.0)
    l2 = jnp.sqrt(jnp.maximum(rsq, 1e-24))
    linf = jnp.maximum(jnp.maximum(jnp.abs(dx), jnp.abs(dy)), jnp.abs(dz))
    scale = jnp.where(linf > 0, l2 / jnp.maximum(linf, 1e-12), 0.0)
    A = []
    for d in (dx, dy, dz):
        co = (d * scale + 1.0) * 1.5
        c0 = jnp.floor(co)
        f = co - c0
        c0i = c0.astype(_i32)
        i0 = jnp.clip(c0i, 0, 3)
        i1 = jnp.clip(c0i + 1, 0, 3)
        A.append([jnp.where(i0 == a, 1.0 - f, 0.0) + jnp.where(i1 == a, f, 0.0)
                  for a in range(4)])
    Ax, Ay, Az = A
    Azw = [az * win for az in Az]
    v = [jnp.where(m, nb[:, 3 + i, :], 0.0) for i in range(3)]
    feat_ref[:, 192:193] = jnp.ones((QB, 1), _f32)
    feat_ref[:, 193:200] = jnp.zeros((QB, 7), _f32)
    for tx in range(4):
        for ty in range(4):
            pxy = Ax[tx] * Ay[ty]
            for tz in range(4):
                w = pxy * Azw[tz]
                row = (tx * 16 + ty * 4 + tz) * 3
                for i in range(3):
                    feat_ref[:, row + i:row + i + 1] = jnp.sum(
                        w * v[i], axis=1, keepdims=True)
    out_ref[...] = jax.lax.dot_general(
        feat_ref[...], wf_ref[...], (((1,), (0,)), ((), ())),
        preferred_element_type=_f32)


def _tc_stage(nbr, cnt, wf):
    return pl.pallas_call(
        _tc_body,
        grid=(NQ // QB,),
        in_specs=[
            pl.BlockSpec((QB, 8, K), lambda i: (i, 0, 0)),
            pl.BlockSpec((QB, 1), lambda i: (i, 0)),
            pl.BlockSpec((200, 32), lambda i: (0, 0)),
        ],
        out_specs=pl.BlockSpec((QB, 32), lambda i: (i, 0)),
        out_shape=jax.ShapeDtypeStruct((NQ, 32), _f32),
        scratch_shapes=[pltpu.VMEM((QB, 200), _f32)],
    )(nbr, cnt, wf)


@jax.jit
def kernel(vel, pos0, pos1, W, b):
    pp = jnp.sum(pos0 * pos0, axis=-1)
    pdata = jnp.concatenate(
        [pos0, vel, pp[:, None], jnp.zeros((N, 1), _f32)], axis=1)
    pdata = jnp.concatenate([pdata, jnp.zeros((NPAD - N, 8), _f32)], axis=0)
    qpos = jnp.concatenate([pos1, jnp.zeros((NQ, 1), _f32)], axis=1)
    qpos = jnp.concatenate([qpos, jnp.zeros((NQPAD - NQ, 4), _f32)], axis=0)
    nbr, cnt = _sc_stage(pdata, qpos)
    wf = jnp.zeros((200, 32), _f32)
    wf = wf.at[:192].set(W.reshape(192, 32)).at[192].set(b)
    return _tc_stage(nbr, cnt.reshape(NQPAD, 1), wf)


# speedup vs baseline: 94.7852x; 1.3377x over previous
import numpy as np

import jax
import jax.numpy as jnp
from jax import lax
from jax.experimental import pallas as pl
from jax.experimental.pallas import tpu as pltpu
from jax.experimental.pallas import tpu_sc as plsc

N = 20000
NQ = 8000
EXT = float(1.0 * 6 * 0.025)
RAD = EXT / 2.0
R2 = float(np.float32(RAD * RAD))
INVR = float(np.float32(2.0 / EXT))

NCELL = 13
OWN = 256
NCPAD = 16 * OWN
CHK = 1280
NPAD = 16 * CHK
NCAP = 4096
NSORT = N + 160
RODMAX = 128
RODCP = 136
K = 128
NQPAD = 8192
QPT = NQPAD // 32
QB = 400

_i32 = jnp.int32
_f32 = jnp.float32


def _sc_body(pdata, qpos, nbr_out, cnt_out,
             sp_cid, sp_cnt, sp_sorted,
             posb, cidall, hitcid, hitpid, perm2d, srows, wo, hist, histv,
             cntf, starts, qposb, qcell, rodbuf, qbuf, cntq, sem):
    s = lax.axis_index("s")
    c = lax.axis_index("c")
    wid = c * 16 + s
    iota = lax.iota(_i32, 16)
    zcol = jnp.zeros((16,), _i32)
    lane0 = iota == 0

    pltpu.sync_copy(pdata.at[pl.ds(s * CHK, CHK)], posb)

    def ph0(j, _):
        rowv = j * 16 + iota
        mrow = (s * CHK + rowv) < N
        rs = jnp.where(mrow, rowv, 0)
        px = plsc.load_gather(posb, [rs, zcol])
        py = plsc.load_gather(posb, [rs, zcol + 1])
        pz = plsc.load_gather(posb, [rs, zcol + 2])
        cx = jnp.clip((px * _f32(NCELL)).astype(_i32), 0, NCELL - 1)
        cy = jnp.clip((py * _f32(NCELL)).astype(_i32), 0, NCELL - 1)
        cz = jnp.clip((pz * _f32(NCELL)).astype(_i32), 0, NCELL - 1)
        cid = (cz * NCELL + cy) * NCELL + cx
        cid = jnp.where(mrow, cid, NCPAD + 99)
        cidall[pl.ds(j * 16, 16)] = cid
        return 0

    lax.fori_loop(0, CHK // 16, ph0, 0)
    pltpu.sync_copy(cidall.at[pl.ds(0, CHK)], sp_cid.at[pl.ds(s * CHK, CHK)])
    plsc.subcore_barrier()

    pltpu.sync_copy(sp_cid, cidall)
    lo = s * OWN

    def ph1(t, off):
        cv = cidall[pl.ds(t * 16, 16)]
        hit = (cv >= lo) & (cv < lo + OWN)
        hi32 = hit.astype(_i32)
        pos = off + plsc.cumsum(hi32) - hi32
        poss = jnp.minimum(pos, NCAP - 1)
        pidx = t * 16 + iota
        plsc.store_scatter(hitcid, [poss], cv, mask=hit)
        plsc.store_scatter(hitpid, [poss], pidx, mask=hit)
        return jnp.minimum(off + jnp.sum(hi32), NCAP)

    nhits = lax.fori_loop(0, NPAD // 16, ph1, 0)

    def zh(j, _):
        hist[j] = 0
        return 0

    lax.fori_loop(0, OWN, zh, 0)

    def ph1b(h, _):
        cl = hitcid[pl.ds(h, 16)][0] - lo
        n0 = hist[cl]
        hist[cl] = n0 + 1
        plsc.store_scatter(wo, [h + iota], jnp.broadcast_to(n0, (16,)),
                           mask=lane0)
        return 0

    lax.fori_loop(0, nhits, ph1b, 0)

    def hcp(j, _):
        hv = jnp.zeros((16,), _i32)
        for u in range(16):
            hv = jnp.where(iota == u, hist[j * 16 + u], hv)
        histv[pl.ds(j * 16, 16)] = hv
        return 0

    lax.fori_loop(0, OWN // 16, hcp, 0)
    pltpu.sync_copy(histv, sp_cnt.at[pl.ds(s * OWN, OWN)])
    plsc.subcore_barrier()

    pltpu.sync_copy(sp_cnt, cntf)

    def ph2(k, carry):
        v = cntf[pl.ds(k * 16, 16)]
        inc = plsc.cumsum(v)
        starts[pl.ds(k * 16, 16)] = carry + inc - v
        return carry + jnp.sum(v)

    total = lax.fori_loop(0, NCPAD // 16, ph2, 0)
    starts[pl.ds(NCPAD, 16)] = jnp.broadcast_to(total, (16,))
    start_lo = starts[pl.ds(lo, 16)][0]

    def pfz(j, _):
        hv = j * 16 + iota
        plsc.store_scatter(perm2d, [hv // 128, hv % 128], zcol)
        return 0

    lax.fori_loop(0, NCAP // 16, pfz, 0)

    def ph2b(j, _):
        hv = j * 16 + iota
        mh = hv < nhits
        hs = jnp.where(mh, hv, 0)
        cidv = plsc.load_gather(hitcid, [hs])
        wov = plsc.load_gather(wo, [hs])
        pidv = plsc.load_gather(hitpid, [hs])
        stv = plsc.load_gather(starts, [jnp.clip(cidv, 0, NCPAD - 1)])
        slot = stv + wov - start_lo
        oks = mh & (slot >= 0) & (slot < NCAP)
        slot = jnp.clip(slot, 0, NCAP - 1)
        plsc.store_scatter(perm2d, [slot // 128, slot % 128], pidv, mask=oks)
        return 0

    lax.fori_loop(0, NCAP // 16, ph2b, 0)

    for chunk in range(NCAP // 128):
        pltpu.async_copy(pdata.at[perm2d.at[chunk]],
                         srows.at[pl.ds(chunk * 128, 128)], sem).wait()

    nch = (nhits + 127) // 128
    shmax = jnp.maximum(nhits - 128, 0)

    def ph2c(j, _):
        offj = jnp.minimum(j * 128, shmax)
        pltpu.sync_copy(srows.at[pl.ds(offj, 128)],
                        sp_sorted.at[pl.ds(start_lo + offj, 128)])
        return 0

    lax.fori_loop(0, nch, ph2c, 0)
    plsc.subcore_barrier()

    qbase = wid * QPT
    pltpu.sync_copy(qpos.at[pl.ds(qbase, QPT)], qposb)

    def ph3cell(j, _):
        rowv = j * 16 + iota
        qx = plsc.load_gather(qposb, [rowv, zcol])
        qy = plsc.load_gather(qposb, [rowv, zcol + 1])
        qz = plsc.load_gather(qposb, [rowv, zcol + 2])
        ccx = jnp.clip((qx * _f32(NCELL)).astype(_i32), 0, NCELL - 1)
        ccy = jnp.clip((qy * _f32(NCELL)).astype(_i32), 0, NCELL - 1)
        ccz = jnp.clip((qz * _f32(NCELL)).astype(_i32), 0, NCELL - 1)
        plsc.store_scatter(qcell, [rowv * 8 + 0], ccx)
        plsc.store_scatter(qcell, [rowv * 8 + 1], ccy)
        plsc.store_scatter(qcell, [rowv * 8 + 2], ccz)
        plsc.store_scatter(qcell, [rowv * 8 + 4], plsc.bitcast(qx, _i32))
        plsc.store_scatter(qcell, [rowv * 8 + 5], plsc.bitcast(qy, _i32))
        plsc.store_scatter(qcell, [rowv * 8 + 6], plsc.bitcast(qz, _i32))
        return 0

    lax.fori_loop(0, QPT // 16, ph3cell, 0)

    def _bf16(x):
        u = plsc.bitcast(x, _i32)
        u2 = (u + 32767 + ((u >> 16) & 1)) & (-65536)
        return plsc.bitcast(u2, _f32)

    def ph3(qi, _):
        crow = qcell[pl.ds(qi * 8, 16)]
        frow = plsc.bitcast(crow, _f32)
        cx = crow[0]
        cy = crow[1]
        cz = crow[2]
        qx = frow[4]
        qy = frow[5]
        qz = frow[6]
        qq = (qx * qx + qy * qy) + qz * qz
        qxb = _bf16(jnp.broadcast_to(qx, (16,)))
        qyb = _bf16(jnp.broadcast_to(qy, (16,)))
        qzb = _bf16(jnp.broadcast_to(qz, (16,)))
        xlo = jnp.maximum(cx - 1, 0)
        xhi = jnp.minimum(cx + 1, NCELL - 1)
        off0 = 0

        rodinfo = []
        descs = []
        for dy in (-1, 0, 1):
            for dz in (-1, 0, 1):
                cy2 = cy + dy
                cz2 = cz + dz
                okrod = (cy2 >= 0) & (cy2 < NCELL) & (cz2 >= 0) & (cz2 < NCELL)
                base = (cz2 * NCELL + cy2) * NCELL
                clo = jnp.clip(base + xlo, 0, NCPAD - 1)
                chi = jnp.clip(base + xhi, 0, NCPAD - 1)
                stt = jnp.clip(starts[pl.ds(clo, 16)][0], 0, N)
                end = starts[pl.ds(chi + 1, 16)][0]
                stt_al = stt & ~7
                skip = stt - stt_al
                n = jnp.where(okrod, jnp.minimum(end - stt, RODMAX), 0)
                r = len(descs)
                descs.append(pltpu.async_copy(
                    sp_sorted.at[pl.ds(stt_al, RODCP)],
                    rodbuf.at[pl.ds(r * RODCP, RODCP)], sem))
                rodinfo.append((skip, n))
        for d in descs:
            d.wait()

        def rod(off, r, skip, n):
            rbase = r * RODCP

            def chunkbody(j, off2):
                lv = j * 16 + iota
                mn = (lv >= skip) & (lv < skip + n)
                ls = jnp.where(mn, lv, 0) + rbase
                px = plsc.load_gather(rodbuf, [ls, zcol])
                py = plsc.load_gather(rodbuf, [ls, zcol + 1])
                pz = plsc.load_gather(rodbuf, [ls, zcol + 2])
                vx = plsc.load_gather(rodbuf, [ls, zcol + 3])
                vy = plsc.load_gather(rodbuf, [ls, zcol + 4])
                vz = plsc.load_gather(rodbuf, [ls, zcol + 5])
                ppv = plsc.load_gather(rodbuf, [ls, zcol + 6])
                ddx = px - qx
                ddy = py - qy
                ddz = pz - qz
                mqp = (qxb * _bf16(px) + qyb * _bf16(py)) + qzb * _bf16(pz)
                d2m = jnp.maximum((qq + ppv) - 2.0 * mqp, 0.0)
                ok = mn & (d2m <= R2) & (d2m > 0.0)
                oki = ok.astype(_i32)
                pos = jnp.minimum(off2 + plsc.cumsum(oki) - oki, K - 1)
                plsc.store_scatter(qbuf, [zcol, pos], ddx, mask=ok)
                plsc.store_scatter(qbuf, [zcol + 1, pos], ddy, mask=ok)
                plsc.store_scatter(qbuf, [zcol + 2, pos], ddz, mask=ok)
                plsc.store_scatter(qbuf, [zcol + 3, pos], vx, mask=ok)
                plsc.store_scatter(qbuf, [zcol + 4, pos], vy, mask=ok)
                plsc.store_scatter(qbuf, [zcol + 5, pos], vz, mask=ok)
                return jnp.minimum(off2 + jnp.sum(oki), K)

            nch = jnp.where(n > 0, (skip + n + 15) // 16, 0)
            return lax.fori_loop(0, nch, chunkbody, off)

        for r, (skip, n) in enumerate(rodinfo):
            off0 = rod(off0, r, skip, n)

        plsc.store_scatter(cntq, [qi + iota],
                           jnp.broadcast_to(off0, (16,)), mask=lane0)
        pltpu.sync_copy(qbuf, nbr_out.at[qbase + qi])
        return 0

    lax.fori_loop(0, QPT, ph3, 0)
    pltpu.sync_copy(cntq.at[pl.ds(0, QPT)], cnt_out.at[pl.ds(qbase, QPT)])


def _sc_stage(pdata, qpos):
    mesh = plsc.VectorSubcoreMesh(core_axis_name="c", subcore_axis_name="s",
                                  num_cores=2, num_subcores=16)
    return pl.kernel(
        _sc_body,
        out_type=[
            jax.ShapeDtypeStruct((NQPAD, 8, K), _f32),
            jax.ShapeDtypeStruct((NQPAD,), _i32),
        ],
        mesh=mesh,
        compiler_params=pltpu.CompilerParams(use_tc_tiling_on_sc=False,
                                             needs_layout_passes=False),
        scratch_types=[
            pltpu.VMEM_SHARED((NPAD,), _i32),
            pltpu.VMEM_SHARED((NCPAD,), _i32),
            pltpu.VMEM_SHARED((NSORT, 8), _f32),
            pltpu.VMEM((CHK, 8), _f32),
            pltpu.VMEM((NPAD,), _i32),
            pltpu.VMEM((NCAP + 16,), _i32),
            pltpu.VMEM((NCAP + 16,), _i32),
            pltpu.VMEM((NCAP // 128, 128), _i32),
            pltpu.VMEM((NCAP, 8), _f32),
            pltpu.VMEM((NCAP + 16,), _i32),
            pltpu.SMEM((OWN,), _i32),
            pltpu.VMEM((OWN,), _i32),
            pltpu.VMEM((NCPAD,), _i32),
            pltpu.VMEM((NCPAD + 32,), _i32),
            pltpu.VMEM((QPT, 4), _f32),
            pltpu.VMEM((QPT * 8 + 16,), _i32),
            pltpu.VMEM((9 * RODCP, 8), _f32),
            pltpu.VMEM((8, K), _f32),
            pltpu.VMEM((QPT + 16,), _i32),
            pltpu.SemaphoreType.DMA,
        ],
    )(pdata, qpos)


def _tc_body(nbr_ref, cnt_ref, wf_ref, out_ref, feat_ref):
    nb = nbr_ref[...]
    cnt = cnt_ref[...]
    lane = lax.broadcasted_iota(_i32, (QB, K), 1)
    m = lane < cnt
    dx = jnp.where(m, nb[:, 0, :] * INVR, 2.0)
    dy = jnp.where(m, nb[:, 1, :] * INVR, 0.0)
    dz = jnp.where(m, nb[:, 2, :] * INVR, 0.0)
    rsq = dx * dx + dy * dy + dz * dz
    t = 1.0 - rsq
    win = jnp.clip(t * t * t, 0.0, 1.0)
    l2 = jnp.sqrt(jnp.maximum(rsq, 1e-24))
    linf = jnp.maximum(jnp.maximum(jnp.abs(dx), jnp.abs(dy)), jnp.abs(dz))
    scale = jnp.where(linf > 0, l2 / jnp.maximum(linf, 1e-12), 0.0)
    A = []
    for d in (dx, dy, dz):
        co = (d * scale + 1.0) * 1.5
        c0 = jnp.floor(co)
        f = co - c0
        c0i = c0.astype(_i32)
        i0 = jnp.clip(c0i, 0, 3)
        i1 = jnp.clip(c0i + 1, 0, 3)
        A.append([jnp.where(i0 == a, 1.0 - f, 0.0) + jnp.where(i1 == a, f, 0.0)
                  for a in range(4)])
    Ax, Ay, Az = A
    Azw = [az * win for az in Az]
    v = [jnp.where(m, nb[:, 3 + i, :], 0.0) for i in range(3)]
    feat_ref[:, 192:193] = jnp.ones((QB, 1), _f32)
    feat_ref[:, 193:200] = jnp.zeros((QB, 7), _f32)
    for tx in range(4):
        for ty in range(4):
            pxy = Ax[tx] * Ay[ty]
            for tz in range(4):
                w = pxy * Azw[tz]
                row = (tx * 16 + ty * 4 + tz) * 3
                for i in range(3):
                    feat_ref[:, row + i:row + i + 1] = jnp.sum(
                        w * v[i], axis=1, keepdims=True)
    out_ref[...] = jax.lax.dot_general(
        feat_ref[...], wf_ref[...], (((1,), (0,)), ((), ())),
        preferred_element_type=_f32)


def _tc_stage(nbr, cnt, wf):
    return pl.pallas_call(
        _tc_body,
        grid=(NQ // QB,),
        in_specs=[
            pl.BlockSpec((QB, 8, K), lambda i: (i, 0, 0)),
            pl.BlockSpec((QB, 1), lambda i: (i, 0)),
            pl.BlockSpec((200, 32), lambda i: (0, 0)),
        ],
        out_specs=pl.BlockSpec((QB, 32), lambda i: (i, 0)),
        out_shape=jax.ShapeDtypeStruct((NQ, 32), _f32),
        scratch_shapes=[pltpu.VMEM((QB, 200), _f32)],
    )(nbr, cnt, wf)


@jax.jit
def kernel(vel, pos0, pos1, W, b):
    pp = jnp.sum(pos0 * pos0, axis=-1)
    pdata = jnp.concatenate(
        [pos0, vel, pp[:, None], jnp.zeros((N, 1), _f32)], axis=1)
    pdata = jnp.concatenate([pdata, jnp.zeros((NPAD - N, 8), _f32)], axis=0)
    qpos = jnp.concatenate([pos1, jnp.zeros((NQ, 1), _f32)], axis=1)
    qpos = jnp.concatenate([qpos, jnp.zeros((NQPAD - NQ, 4), _f32)], axis=0)
    nbr, cnt = _sc_stage(pdata, qpos)
    wf = jnp.zeros((200, 32), _f32)
    wf = wf.at[:192].set(W.reshape(192, 32)).at[192].set(b)
    return _tc_stage(nbr, cnt.reshape(NQPAD, 1), wf)


# speedup vs baseline: 98.8641x; 1.0430x over previous
import numpy as np

import jax
import jax.numpy as jnp
from jax import lax
from jax.experimental import pallas as pl
from jax.experimental.pallas import tpu as pltpu
from jax.experimental.pallas import tpu_sc as plsc

N = 20000
NQ = 8000
EXT = float(1.0 * 6 * 0.025)
RAD = EXT / 2.0
R2 = float(np.float32(RAD * RAD))
INVR = float(np.float32(2.0 / EXT))

NCELL = 13
OWN = 256
NCPAD = 16 * OWN
CHK = 1280
NPAD = 16 * CHK
NCAP = 4096
NSORT = N + 160
RODMAX = 128
RODCP = 136
K = 128
NQPAD = 8192
QPT = NQPAD // 32
QB = 400

_i32 = jnp.int32
_f32 = jnp.float32


def _sc_body(pdata, qpos, nbr_out, cnt_out,
             sp_cid, sp_cnt, sp_sorted,
             posb, cidall, hitcid, hitpid, perm2d, srows, wo, hist, histv,
             cntf, starts, qposb, qcell, rodbuf, qbuf, cntq, sem):
    s = lax.axis_index("s")
    c = lax.axis_index("c")
    wid = c * 16 + s
    iota = lax.iota(_i32, 16)
    zcol = jnp.zeros((16,), _i32)
    lane0 = iota == 0

    pltpu.sync_copy(pdata.at[pl.ds(s * CHK, CHK)], posb)

    def ph0(j, _):
        rowv = j * 16 + iota
        mrow = (s * CHK + rowv) < N
        rs = jnp.where(mrow, rowv, 0)
        px = plsc.load_gather(posb, [rs, zcol])
        py = plsc.load_gather(posb, [rs, zcol + 1])
        pz = plsc.load_gather(posb, [rs, zcol + 2])
        cx = jnp.clip((px * _f32(NCELL)).astype(_i32), 0, NCELL - 1)
        cy = jnp.clip((py * _f32(NCELL)).astype(_i32), 0, NCELL - 1)
        cz = jnp.clip((pz * _f32(NCELL)).astype(_i32), 0, NCELL - 1)
        cid = (cz * NCELL + cy) * NCELL + cx
        cid = jnp.where(mrow, cid, NCPAD + 99)
        cidall[pl.ds(j * 16, 16)] = cid
        return 0

    lax.fori_loop(0, CHK // 16, ph0, 0)
    pltpu.sync_copy(cidall.at[pl.ds(0, CHK)], sp_cid.at[pl.ds(s * CHK, CHK)])
    plsc.subcore_barrier()

    pltpu.sync_copy(sp_cid, cidall)
    lo = s * OWN

    def ph1(t, off):
        cv = cidall[pl.ds(t * 16, 16)]
        hit = (cv >= lo) & (cv < lo + OWN)
        hi32 = hit.astype(_i32)
        pos = off + plsc.cumsum(hi32) - hi32
        poss = jnp.minimum(pos, NCAP - 1)
        pidx = t * 16 + iota
        plsc.store_scatter(hitcid, [poss], cv, mask=hit)
        plsc.store_scatter(hitpid, [poss], pidx, mask=hit)
        return jnp.minimum(off + jnp.sum(hi32), NCAP)

    nhits = lax.fori_loop(0, NPAD // 16, ph1, 0)

    def zh(j, _):
        hist[j] = 0
        return 0

    lax.fori_loop(0, OWN, zh, 0)

    def ph1b(h, _):
        cl = hitcid[pl.ds(h, 16)][0] - lo
        n0 = hist[cl]
        hist[cl] = n0 + 1
        plsc.store_scatter(wo, [h + iota], jnp.broadcast_to(n0, (16,)),
                           mask=lane0)
        return 0

    lax.fori_loop(0, nhits, ph1b, 0)

    def hcp(j, _):
        hv = jnp.zeros((16,), _i32)
        for u in range(16):
            hv = jnp.where(iota == u, hist[j * 16 + u], hv)
        histv[pl.ds(j * 16, 16)] = hv
        return 0

    lax.fori_loop(0, OWN // 16, hcp, 0)
    pltpu.sync_copy(histv, sp_cnt.at[pl.ds(s * OWN, OWN)])
    plsc.subcore_barrier()

    pltpu.sync_copy(sp_cnt, cntf)

    def ph2(k, carry):
        v = cntf[pl.ds(k * 16, 16)]
        inc = plsc.cumsum(v)
        starts[pl.ds(k * 16, 16)] = carry + inc - v
        return carry + jnp.sum(v)

    total = lax.fori_loop(0, NCPAD // 16, ph2, 0)
    starts[pl.ds(NCPAD, 16)] = jnp.broadcast_to(total, (16,))
    start_lo = starts[pl.ds(lo, 16)][0]

    def pfz(j, _):
        hv = j * 16 + iota
        plsc.store_scatter(perm2d, [hv // 128, hv % 128], zcol)
        return 0

    lax.fori_loop(0, NCAP // 16, pfz, 0)

    def ph2b(j, _):
        hv = j * 16 + iota
        mh = hv < nhits
        hs = jnp.where(mh, hv, 0)
        cidv = plsc.load_gather(hitcid, [hs])
        wov = plsc.load_gather(wo, [hs])
        pidv = plsc.load_gather(hitpid, [hs])
        stv = plsc.load_gather(starts, [jnp.clip(cidv, 0, NCPAD - 1)])
        slot = stv + wov - start_lo
        oks = mh & (slot >= 0) & (slot < NCAP)
        slot = jnp.clip(slot, 0, NCAP - 1)
        plsc.store_scatter(perm2d, [slot // 128, slot % 128], pidv, mask=oks)
        return 0

    lax.fori_loop(0, NCAP // 16, ph2b, 0)

    for chunk in range(NCAP // 128):
        pltpu.async_copy(pdata.at[perm2d.at[chunk]],
                         srows.at[pl.ds(chunk * 128, 128)], sem).wait()

    nch = (nhits + 127) // 128
    shmax = jnp.maximum(nhits - 128, 0)

    def ph2c(j, _):
        offj = jnp.minimum(j * 128, shmax)
        pltpu.sync_copy(srows.at[pl.ds(offj, 128)],
                        sp_sorted.at[pl.ds(start_lo + offj, 128)])
        return 0

    lax.fori_loop(0, nch, ph2c, 0)
    plsc.subcore_barrier()

    qbase = wid * QPT
    pltpu.sync_copy(qpos.at[pl.ds(qbase, QPT)], qposb)

    def ph3cell(j, _):
        rowv = j * 16 + iota
        qx = plsc.load_gather(qposb, [rowv, zcol])
        qy = plsc.load_gather(qposb, [rowv, zcol + 1])
        qz = plsc.load_gather(qposb, [rowv, zcol + 2])
        ccx = jnp.clip((qx * _f32(NCELL)).astype(_i32), 0, NCELL - 1)
        ccy = jnp.clip((qy * _f32(NCELL)).astype(_i32), 0, NCELL - 1)
        ccz = jnp.clip((qz * _f32(NCELL)).astype(_i32), 0, NCELL - 1)
        plsc.store_scatter(qcell, [rowv * 8 + 0], ccx)
        plsc.store_scatter(qcell, [rowv * 8 + 1], ccy)
        plsc.store_scatter(qcell, [rowv * 8 + 2], ccz)
        plsc.store_scatter(qcell, [rowv * 8 + 4], plsc.bitcast(qx, _i32))
        plsc.store_scatter(qcell, [rowv * 8 + 5], plsc.bitcast(qy, _i32))
        plsc.store_scatter(qcell, [rowv * 8 + 6], plsc.bitcast(qz, _i32))
        return 0

    lax.fori_loop(0, QPT // 16, ph3cell, 0)

    def _bf16(x):
        u = plsc.bitcast(x, _i32)
        u2 = (u + 32767 + ((u >> 16) & 1)) & (-65536)
        return plsc.bitcast(u2, _f32)

    def ph3(qi, _):
        crow = qcell[pl.ds(qi * 8, 16)]
        frow = plsc.bitcast(crow, _f32)
        cx = crow[0]
        cy = crow[1]
        cz = crow[2]
        qx = frow[4]
        qy = frow[5]
        qz = frow[6]
        qq = (qx * qx + qy * qy) + qz * qz
        qxb = _bf16(jnp.broadcast_to(qx, (16,)))
        qyb = _bf16(jnp.broadcast_to(qy, (16,)))
        qzb = _bf16(jnp.broadcast_to(qz, (16,)))
        xlo = jnp.maximum(cx - 1, 0)
        xhi = jnp.minimum(cx + 1, NCELL - 1)
        off0 = 0

        rodinfo = []
        descs = []
        for dy in (-1, 0, 1):
            for dz in (-1, 0, 1):
                cy2 = cy + dy
                cz2 = cz + dz
                okrod = (cy2 >= 0) & (cy2 < NCELL) & (cz2 >= 0) & (cz2 < NCELL)
                base = (cz2 * NCELL + cy2) * NCELL
                clo = jnp.clip(base + xlo, 0, NCPAD - 1)
                chi = jnp.clip(base + xhi, 0, NCPAD - 1)
                stt = jnp.clip(starts[pl.ds(clo, 16)][0], 0, N)
                end = starts[pl.ds(chi + 1, 16)][0]
                stt_al = stt & ~7
                skip = stt - stt_al
                n = jnp.where(okrod, jnp.minimum(end - stt, RODMAX), 0)
                r = len(descs)
                descs.append(pltpu.async_copy(
                    sp_sorted.at[pl.ds(stt_al, RODCP)],
                    rodbuf.at[pl.ds(r * RODCP, RODCP)], sem))
                rodinfo.append((skip, n))

        def rod(off, r, skip, n):
            rbase = r * RODCP

            def chunkbody(j, off2):
                lv = j * 16 + iota
                mn = (lv >= skip) & (lv < skip + n)
                ls = jnp.where(mn, lv, 0) + rbase
                px = plsc.load_gather(rodbuf, [ls, zcol])
                py = plsc.load_gather(rodbuf, [ls, zcol + 1])
                pz = plsc.load_gather(rodbuf, [ls, zcol + 2])
                vx = plsc.load_gather(rodbuf, [ls, zcol + 3])
                vy = plsc.load_gather(rodbuf, [ls, zcol + 4])
                vz = plsc.load_gather(rodbuf, [ls, zcol + 5])
                ppv = plsc.load_gather(rodbuf, [ls, zcol + 6])
                ddx = px - qx
                ddy = py - qy
                ddz = pz - qz
                mqp = (qxb * _bf16(px) + qyb * _bf16(py)) + qzb * _bf16(pz)
                d2m = jnp.maximum((qq + ppv) - 2.0 * mqp, 0.0)
                ok = mn & (d2m <= R2) & (d2m > 0.0)
                oki = ok.astype(_i32)
                pos = jnp.minimum(off2 + plsc.cumsum(oki) - oki, K - 1)
                plsc.store_scatter(qbuf, [zcol, pos], ddx, mask=ok)
                plsc.store_scatter(qbuf, [zcol + 1, pos], ddy, mask=ok)
                plsc.store_scatter(qbuf, [zcol + 2, pos], ddz, mask=ok)
                plsc.store_scatter(qbuf, [zcol + 3, pos], vx, mask=ok)
                plsc.store_scatter(qbuf, [zcol + 4, pos], vy, mask=ok)
                plsc.store_scatter(qbuf, [zcol + 5, pos], vz, mask=ok)
                return jnp.minimum(off2 + jnp.sum(oki), K)

            nch = jnp.where(n > 0, (skip + n + 15) // 16, 0)
            return lax.fori_loop(0, nch, chunkbody, off)

        for r, (skip, n) in enumerate(rodinfo):
            descs[r].wait()
            off0 = rod(off0, r, skip, n)

        plsc.store_scatter(cntq, [qi + iota],
                           jnp.broadcast_to(off0, (16,)), mask=lane0)
        pltpu.sync_copy(qbuf, nbr_out.at[qbase + qi])
        return 0

    lax.fori_loop(0, QPT, ph3, 0)
    pltpu.sync_copy(cntq.at[pl.ds(0, QPT)], cnt_out.at[pl.ds(qbase, QPT)])


def _sc_stage(pdata, qpos):
    mesh = plsc.VectorSubcoreMesh(core_axis_name="c", subcore_axis_name="s",
                                  num_cores=2, num_subcores=16)
    return pl.kernel(
        _sc_body,
        out_type=[
            jax.ShapeDtypeStruct((NQPAD, 8, K), _f32),
            jax.ShapeDtypeStruct((NQPAD,), _i32),
        ],
        mesh=mesh,
        compiler_params=pltpu.CompilerParams(use_tc_tiling_on_sc=False,
                                             needs_layout_passes=False),
        scratch_types=[
            pltpu.VMEM_SHARED((NPAD,), _i32),
            pltpu.VMEM_SHARED((NCPAD,), _i32),
            pltpu.VMEM_SHARED((NSORT, 8), _f32),
            pltpu.VMEM((CHK, 8), _f32),
            pltpu.VMEM((NPAD,), _i32),
            pltpu.VMEM((NCAP + 16,), _i32),
            pltpu.VMEM((NCAP + 16,), _i32),
            pltpu.VMEM((NCAP // 128, 128), _i32),
            pltpu.VMEM((NCAP, 8), _f32),
            pltpu.VMEM((NCAP + 16,), _i32),
            pltpu.SMEM((OWN,), _i32),
            pltpu.VMEM((OWN,), _i32),
            pltpu.VMEM((NCPAD,), _i32),
            pltpu.VMEM((NCPAD + 32,), _i32),
            pltpu.VMEM((QPT, 4), _f32),
            pltpu.VMEM((QPT * 8 + 16,), _i32),
            pltpu.VMEM((9 * RODCP, 8), _f32),
            pltpu.VMEM((8, K), _f32),
            pltpu.VMEM((QPT + 16,), _i32),
            pltpu.SemaphoreType.DMA,
        ],
    )(pdata, qpos)


def _tc_body(nbr_ref, cnt_ref, wf_ref, out_ref, feat_ref):
    nb = nbr_ref[...]
    cnt = cnt_ref[...]
    lane = lax.broadcasted_iota(_i32, (QB, K), 1)
    m = lane < cnt
    dx = jnp.where(m, nb[:, 0, :] * INVR, 2.0)
    dy = jnp.where(m, nb[:, 1, :] * INVR, 0.0)
    dz = jnp.where(m, nb[:, 2, :] * INVR, 0.0)
    rsq = dx * dx + dy * dy + dz * dz
    t = 1.0 - rsq
    win = jnp.clip(t * t * t, 0.0, 1.0)
    l2 = jnp.sqrt(jnp.maximum(rsq, 1e-24))
    linf = jnp.maximum(jnp.maximum(jnp.abs(dx), jnp.abs(dy)), jnp.abs(dz))
    scale = jnp.where(linf > 0, l2 / jnp.maximum(linf, 1e-12), 0.0)
    A = []
    for d in (dx, dy, dz):
        co = (d * scale + 1.0) * 1.5
        c0 = jnp.floor(co)
        f = co - c0
        c0i = c0.astype(_i32)
        i0 = jnp.clip(c0i, 0, 3)
        i1 = jnp.clip(c0i + 1, 0, 3)
        A.append([jnp.where(i0 == a, 1.0 - f, 0.0) + jnp.where(i1 == a, f, 0.0)
                  for a in range(4)])
    Ax, Ay, Az = A
    Azw = [az * win for az in Az]
    v = [jnp.where(m, nb[:, 3 + i, :], 0.0) for i in range(3)]
    feat_ref[:, 192:193] = jnp.ones((QB, 1), _f32)
    feat_ref[:, 193:200] = jnp.zeros((QB, 7), _f32)
    for tx in range(4):
        for ty in range(4):
            pxy = Ax[tx] * Ay[ty]
            for tz in range(4):
                w = pxy * Azw[tz]
                row = (tx * 16 + ty * 4 + tz) * 3
                for i in range(3):
                    feat_ref[:, row + i:row + i + 1] = jnp.sum(
                        w * v[i], axis=1, keepdims=True)
    out_ref[...] = jax.lax.dot_general(
        feat_ref[...], wf_ref[...], (((1,), (0,)), ((), ())),
        preferred_element_type=_f32)


def _tc_stage(nbr, cnt, wf):
    return pl.pallas_call(
        _tc_body,
        grid=(NQ // QB,),
        in_specs=[
            pl.BlockSpec((QB, 8, K), lambda i: (i, 0, 0)),
            pl.BlockSpec((QB, 1), lambda i: (i, 0)),
            pl.BlockSpec((200, 32), lambda i: (0, 0)),
        ],
        out_specs=pl.BlockSpec((QB, 32), lambda i: (i, 0)),
        out_shape=jax.ShapeDtypeStruct((NQ, 32), _f32),
        scratch_shapes=[pltpu.VMEM((QB, 200), _f32)],
    )(nbr, cnt, wf)


@jax.jit
def kernel(vel, pos0, pos1, W, b):
    pp = jnp.sum(pos0 * pos0, axis=-1)
    pdata = jnp.concatenate(
        [pos0, vel, pp[:, None], jnp.zeros((N, 1), _f32)], axis=1)
    pdata = jnp.concatenate([pdata, jnp.zeros((NPAD - N, 8), _f32)], axis=0)
    qpos = jnp.concatenate([pos1, jnp.zeros((NQ, 1), _f32)], axis=1)
    qpos = jnp.concatenate([qpos, jnp.zeros((NQPAD - NQ, 4), _f32)], axis=0)
    nbr, cnt = _sc_stage(pdata, qpos)
    wf = jnp.zeros((200, 32), _f32)
    wf = wf.at[:192].set(W.reshape(192, 32)).at[192].set(b)
    return _tc_stage(nbr, cnt.reshape(NQPAD, 1), wf)
